# Initial kernel scaffold; baseline (speedup 1.0000x reference)
import jax
import jax.numpy as jnp
from jax.experimental import pallas as pl


def kernel(X):
    raise NotImplementedError("write your pallas kernel here")


# speedup vs baseline: 4.2495x; 4.2495x over previous
import functools

import jax
import jax.numpy as jnp
from jax import lax
from jax.experimental import pallas as pl
from jax.experimental.pallas import tpu as pltpu
from jax.experimental.pallas import tpu_sc as plsc

L = 16
R = 1024
K = 16
N = 256
G = R // L
NW = 32
SPW = N // NW
ITERS = 10
F32 = jnp.float32


def _r16(x):
    bits = jax.lax.bitcast_convert_type(x, jnp.uint32)
    sign = bits & jnp.uint32(0x80000000)
    mag = bits & jnp.uint32(0x7FFFFFFF)
    odd = (mag >> 13) & jnp.uint32(1)
    rb = (mag + jnp.uint32(0xFFF) + odd) & jnp.uint32(0xFFFFE000)
    xn = jax.lax.bitcast_convert_type(sign | rb, jnp.float32)
    ax = jnp.abs(x)
    xs = (ax + jnp.float32(0.5)) - jnp.float32(0.5)
    xs = jnp.where(x < 0, -xs, xs)
    return jnp.where(ax >= jnp.float32(2.0 ** -14), xn, xs)


def _rbf(x):
    bits = jax.lax.bitcast_convert_type(x, jnp.uint32)
    odd = (bits >> 16) & jnp.uint32(1)
    rb = (bits + jnp.uint32(0x7FFF) + odd) & jnp.uint32(0xFFFF0000)
    return jax.lax.bitcast_convert_type(rb, jnp.float32)


def _sqrt(x):
    bits = jax.lax.bitcast_convert_type(x, jnp.uint32)
    s = jax.lax.bitcast_convert_type(
        (bits >> 1) + jnp.uint32(0x1FBD1DF5), jnp.float32)
    for _ in range(3):
        s = 0.5 * (s + x / s)
    return jnp.where(x > 0.0, s, 0.0)


def _lane(vec, off, iota):
    return jnp.sum(jnp.where(iota == off, vec, 0.0))


def _kpp_v():
    key = jax.random.key(42)
    vs = []
    for _b in range(N // 8):
        for _i in range(K - 1):
            key, sk = jax.random.split(key)
            vs.append(jax.random.uniform(sk, (8, 1), dtype=jnp.float32))
    V = jnp.stack(vs, 0).reshape(N // 8, K - 1, 8)
    V = jnp.transpose(V, (0, 2, 1)).reshape(N, K - 1)
    return jnp.concatenate([jnp.zeros((N, 1), F32), V], axis=1)


def _sc_call(x0f, x1f, vf):
    mesh = plsc.VectorSubcoreMesh(core_axis_name="c", subcore_axis_name="s")

    @functools.partial(
        pl.kernel,
        out_type=jax.ShapeDtypeStruct((N * 2 * K,), F32),
        mesh=mesh,
        compiler_params=pltpu.CompilerParams(needs_layout_passes=False),
        scratch_types=[
            pltpu.VMEM((R,), F32),
            pltpu.VMEM((R,), F32),
            pltpu.VMEM((R,), F32),
            pltpu.VMEM((R,), F32),
            pltpu.VMEM((R,), F32),
            pltpu.VMEM((R,), F32),
            pltpu.VMEM((K,), F32),
            pltpu.VMEM((L,), F32),
            pltpu.VMEM((L,), F32),
            pltpu.VMEM((L,), F32),
            pltpu.VMEM((L,), F32),
            pltpu.VMEM((L,), F32),
            pltpu.VMEM((2 * K,), F32),
        ],
    )
    def k(x0h, x1h, vh, outh, x0, x1, b0, b1, an, rm, vbuf, s0r, s1r, ctr,
          cs0, cs1, ob):
        cidx = lax.axis_index("c")
        sidx = lax.axis_index("s")
        wid = sidx * 2 + cidx
        iota = lax.iota(jnp.int32, L)
        zf = jnp.zeros((L,), F32)
        zi = jnp.zeros((L,), jnp.int32)

        def per_sub(jj, _):
            n = wid * SPW + jj
            pltpu.sync_copy(x0h.at[pl.ds(n * R, R)], x0)
            pltpu.sync_copy(x1h.at[pl.ds(n * R, R)], x1)
            pltpu.sync_copy(vh.at[pl.ds(n * K, K)], vbuf)

            def g0(g, _c):
                sl = pl.ds(g * L, L)
                a0 = x0[sl]
                a1 = x1[sl]
                b0[sl] = _rbf(_r16(a0))
                b1[sl] = _rbf(_r16(a1))
                an[sl] = a0 * a0 + a1 * a1
                return 0

            lax.fori_loop(0, G, g0, 0)

            xa0 = x0[pl.ds(0, L)][0]
            xa1 = x1[pl.ds(0, L)][0]
            ba0 = b0[pl.ds(0, L)][0]
            ba1 = b1[pl.ds(0, L)][0]
            ana = xa0 * xa0 + xa1 * xa1

            def g1(g, _c):
                sl = pl.ds(g * L, L)
                d2 = (ana + an[sl]) - 2.0 * (ba0 * b0[sl] + ba1 * b1[sl])
                rm[sl] = _r16(_sqrt(jnp.maximum(d2, 0.0)))
                return 0

            lax.fori_loop(0, G, g1, 0)
            c0v = jnp.where(iota == 0, _r16(zf + xa0), zf)
            c1v = jnp.where(iota == 0, _r16(zf + xa1), zf)

            def step(i, carry):
                c0v, c1v = carry
                v_i = _lane(vbuf[pl.ds(0, L)], i, iota)

                def gs(g, acc):
                    return acc + jnp.sum(rm[pl.ds(g * L, L)])

                s = lax.fori_loop(0, G, gs, jnp.float32(0.0))

                def gc(g, carry2):
                    acc, cnt = carry2
                    q = rm[pl.ds(g * L, L)] / s
                    cum = jnp.cumsum(q) + acc
                    cnt = cnt + jnp.sum((cum < v_i).astype(jnp.int32))
                    return acc + jnp.sum(q), cnt

                _, cnt_s = lax.fori_loop(
                    0, G, gc, (jnp.float32(0.0), jnp.int32(0)))
                idx_s = jnp.minimum(cnt_s, R - 1)
                base = jnp.left_shift(jnp.right_shift(idx_s, 4), 4)
                off = idx_s - base
                xb0 = _lane(x0[pl.ds(base, L)], off, iota)
                xb1 = _lane(x1[pl.ds(base, L)], off, iota)
                bb0 = _lane(b0[pl.ds(base, L)], off, iota)
                bb1 = _lane(b1[pl.ds(base, L)], off, iota)
                anb = _lane(an[pl.ds(base, L)], off, iota)

                def gd(g, _c):
                    sl = pl.ds(g * L, L)
                    d2 = (anb + an[sl]) - 2.0 * (bb0 * b0[sl] + bb1 * b1[sl])
                    rm[sl] = jnp.minimum(
                        rm[sl], _r16(_sqrt(jnp.maximum(d2, 0.0))))
                    return 0

                lax.fori_loop(0, G, gd, 0)
                c0v = jnp.where(iota == i, _r16(zf + xb0), c0v)
                c1v = jnp.where(iota == i, _r16(zf + xb1), c1v)
                return c0v, c1v

            c0v, c1v = lax.fori_loop(1, K, step, (c0v, c1v))

            def lloyd(t, carry):
                c0v, c1v = carry
                c0s = [_lane(c0v, kk, iota) for kk in range(K)]
                c1s = [_lane(c1v, kk, iota) for kk in range(K)]

                def ga(g, accs):
                    s0a, s1a, ca = accs
                    sl = pl.ds(g * L, L)
                    a0 = x0[sl]
                    a1 = x1[sl]
                    d0 = a0 - c0s[0]
                    d1 = a1 - c1s[0]
                    best = d0 * d0 + d1 * d1
                    bk = zi
                    for kk in range(1, K):
                        e0 = a0 - c0s[kk]
                        e1 = a1 - c1s[kk]
                        dd = e0 * e0 + e1 * e1
                        m = dd < best
                        best = jnp.where(m, dd, best)
                        bk = jnp.where(m, kk, bk)
                    s0a = tuple(s0a[kk] + jnp.where(bk == kk, a0, 0.0)
                                for kk in range(K))
                    s1a = tuple(s1a[kk] + jnp.where(bk == kk, a1, 0.0)
                                for kk in range(K))
                    ca = tuple(ca[kk] + jnp.where(bk == kk, 1.0, 0.0)
                               for kk in range(K))
                    return s0a, s1a, ca

                z16 = tuple(zf for _ in range(K))
                s0a, s1a, ca = lax.fori_loop(0, G, ga, (z16, z16, z16))
                s0v = zf
                s1v = zf
                cv = zf
                for kk in range(K):
                    s0v = jnp.where(iota == kk, jnp.sum(s0a[kk]), s0v)
                    s1v = jnp.where(iota == kk, jnp.sum(s1a[kk]), s1v)
                    cv = jnp.where(iota == kk, jnp.sum(ca[kk]), cv)
                normv = 1.0 / jnp.maximum(cv, 1.0)
                return _r16(s0v * normv), _r16(s1v * normv)

            c0v, c1v = lax.fori_loop(0, ITERS, lloyd, (c0v, c1v))
            ob[pl.ds(0, L)] = c0v
            ob[pl.ds(L, L)] = c1v
            pltpu.sync_copy(ob, outh.at[pl.ds(n * 2 * K, 2 * K)])
            return 0

        lax.fori_loop(0, SPW, per_sub, 0)

    return k(x0f, x1f, vf)


def kernel(X):
    Xr = jnp.transpose(X.reshape(R, N, 2), (1, 0, 2))
    x0f = Xr[:, :, 0].reshape(-1)
    x1f = Xr[:, :, 1].reshape(-1)
    V = _kpp_v()
    out = _sc_call(x0f, x1f, V.reshape(-1))
    out = out.reshape(N, 2, K).transpose(0, 2, 1)
    return out.astype(jnp.float16)
